# baseline (speedup 1.0000x reference)
import functools

import jax
import jax.numpy as jnp
from jax import lax
from jax.experimental import pallas as pl
from jax.experimental.pallas import tpu as pltpu
from jax.experimental.pallas import tpu_sc as plsc

N = 10000
E = 50000
DIN = 256
DOUT = 32
DE = 16
HNW = DOUT

NC = 2
NS = 16
NW = NC * NS
CHUNK = 128
QPC = CHUNK // 4
CH_PER_TILE = 13
E_TILE = CHUNK * CH_PER_TILE
E_PAD = NW * E_TILE
N_PAD = 10240
ROWS_TILE = N_PAD // NS
AW = 48

_SC_MESH = plsc.VectorSubcoreMesh(
    core_axis_name="c", subcore_axis_name="s", num_cores=NC, num_subcores=NS)


@functools.partial(
    pl.kernel,
    out_type=jax.ShapeDtypeStruct((NC, N_PAD, AW), jnp.float32),
    mesh=_SC_MESH,
    compiler_params=pltpu.CompilerParams(use_tc_tiling_on_sc=False),
    scratch_types=[
        pltpu.VMEM((CH_PER_TILE, 1, CHUNK), jnp.int32),
        pltpu.VMEM((CH_PER_TILE, 1, CHUNK), jnp.int32),
        pltpu.VMEM((E_TILE, DOUT), jnp.float32),
        pltpu.VMEM((CHUNK, HNW), jnp.float32),
        pltpu.VMEM((CHUNK, HNW), jnp.float32),
        pltpu.VMEM((CHUNK, AW), jnp.float32),
        pltpu.VMEM((CHUNK, AW), jnp.float32),
        pltpu.VMEM_SHARED((N_PAD, AW), jnp.float32),
        pltpu.SemaphoreType.DMA,
        pltpu.SemaphoreType.DMA,
        pltpu.SemaphoreType.DMA,
        pltpu.SemaphoreType.DMA,
        pltpu.SemaphoreType.DMA,
    ],
)
def _sc_edge_scatter(hn_hbm, src_hbm, dst_hbm, ew_hbm, out_hbm,
                     idx_src, idx_dst, ew_all, rows0, rows1, msg0, msg1,
                     acc_sp, esem, gsem0, gsem1, ssem0, ssem1):
    c = lax.axis_index("c")
    s = lax.axis_index("s")
    wid = c * NS + s

    rows = (rows0, rows1)
    msg = (msg0, msg1)
    gsem = (gsem0, gsem1)
    ssem = (ssem0, ssem1)

    ew_cp = pltpu.async_copy(
        ew_hbm.at[pl.ds(wid * E_TILE, E_TILE)], ew_all, esem)
    pltpu.sync_copy(src_hbm.at[pl.ds(wid * CH_PER_TILE, CH_PER_TILE)], idx_src)
    pltpu.sync_copy(dst_hbm.at[pl.ds(wid * CH_PER_TILE, CH_PER_TILE)], idx_dst)

    zeros16 = jnp.zeros((16,), jnp.float32)

    def _zero_row(i, _):
        msg0[i, pl.ds(0, 16)] = zeros16
        msg0[i, pl.ds(16, 16)] = zeros16
        msg0[i, pl.ds(32, 16)] = zeros16
        return 0
    lax.fori_loop(0, CHUNK, _zero_row, 0)

    def _zero_acc(j, _):
        pltpu.sync_copy(msg0, acc_sp.at[pl.ds(s * ROWS_TILE + j * CHUNK, CHUNK)])
        return 0
    lax.fori_loop(0, ROWS_TILE // CHUNK, _zero_acc, 0)

    deg_lane = jnp.where(lax.iota(jnp.int32, 16) == 0,
                         jnp.float32(1.0), jnp.float32(0.0))

    def _set_deg(i, _):
        msg0[i, pl.ds(DOUT, 16)] = deg_lane
        msg1[i, pl.ds(DOUT, 16)] = deg_lane
        return 0
    lax.fori_loop(0, CHUNK, _set_deg, 0)

    ew_cp.wait()
    plsc.subcore_barrier()

    gcp = [None, None]
    scp = [None, None]
    gcp[0] = pltpu.async_copy(hn_hbm.at[idx_src.at[0, 0]], rows0, gsem0)
    for j in range(CH_PER_TILE):
        b = j & 1
        nb = b ^ 1
        if j + 1 < CH_PER_TILE:
            gcp[nb] = pltpu.async_copy(
                hn_hbm.at[idx_src.at[j + 1, 0]], rows[nb], gsem[nb])
        gcp[b].wait()
        if scp[b] is not None:
            scp[b].wait()
        rb = rows[b]
        mb = msg[b]

        e0 = j * CHUNK

        def _mul(q, _):
            r = q * 4
            for k in range(4):
                for h in range(2):
                    mb[r + k, pl.ds(h * 16, 16)] = (
                        rb[r + k, pl.ds(h * 16, 16)]
                        * ew_all[e0 + r + k, pl.ds(h * 16, 16)])
            return 0
        lax.fori_loop(0, QPC, _mul, 0)
        scp[b] = pltpu.async_copy(mb, acc_sp.at[idx_dst.at[j, 0]], ssem[b], add=True)
    scp[0].wait()
    scp[1].wait()
    plsc.subcore_barrier()

    def _export(j, _):
        r0 = s * ROWS_TILE + j * CHUNK
        pltpu.sync_copy(acc_sp.at[pl.ds(r0, CHUNK)], msg0)
        pltpu.sync_copy(msg0, out_hbm.at[c, pl.ds(r0, CHUNK)])
        return 0
    lax.fori_loop(0, ROWS_TILE // CHUNK, _export, 0)


def _preagg_body(x_ref, w_ref, o_ref):
    y = lax.dot_general(x_ref[...], w_ref[...], (((1,), (1,)), ((), ())),
                        preferred_element_type=jnp.float32)
    o_ref[...] = jnp.maximum(y, 0.0)


def _edge_body(ef_ref, w_ref, b_ref, o_ref):
    y = lax.dot_general(ef_ref[...].astype(jnp.bfloat16), w_ref[...],
                        (((1,), (1,)), ((), ())),
                        preferred_element_type=jnp.float32)
    y = jnp.maximum(y + b_ref[...], 0.0)
    for half in (512, 256, 128, 64, 32):
        y = y[:, :half] + y[:, half:2 * half]
    o_ref[...] = y


def _final_body(hs_ref, acc_ref, wp_ref, ws_ref, wn_ref, o_ref):
    hs = jnp.maximum(
        lax.dot_general(hs_ref[...], wp_ref[...], (((1,), (1,)), ((), ())),
                        preferred_element_type=jnp.float32), 0.0)
    a = acc_ref[0] + acc_ref[1]
    neigh = a[:, :DOUT] / jnp.maximum(a[:, DOUT:DOUT + 1], 1.0)
    z1 = jnp.maximum(
        lax.dot_general(hs, ws_ref[...], (((1,), (1,)), ((), ())),
                        preferred_element_type=jnp.float32), 0.0)
    z2 = jnp.maximum(
        lax.dot_general(neigh, wn_ref[...], (((1,), (1,)), ((), ())),
                        preferred_element_type=jnp.float32), 0.0)
    o_ref[...] = jnp.maximum(z1 + z2, 0.0)


def kernel(h_neigh, h_self, edge_features, W_preagg, W_self, W_neigh,
           W_edge, b_edge, edge_index):
    src = edge_index[0]
    dst = edge_index[1]
    npad = E_PAD - E
    pad_dst = N + (jnp.arange(npad, dtype=jnp.int32) % (N_PAD - N))
    pad_src = jnp.arange(npad, dtype=jnp.int32) % N
    src_pad = jnp.concatenate(
        [src, pad_src]).reshape(NW * CH_PER_TILE, 1, CHUNK)
    dst_pad = jnp.concatenate(
        [dst, pad_dst]).reshape(NW * CH_PER_TILE, 1, CHUNK)
    hn = pl.pallas_call(
        _preagg_body,
        grid=(5,),
        in_specs=[pl.BlockSpec((2000, DIN), lambda i: (i, 0)),
                  pl.BlockSpec((HNW, DIN), lambda i: (0, 0))],
        out_specs=pl.BlockSpec((2000, HNW), lambda i: (i, 0)),
        out_shape=jax.ShapeDtypeStruct((N, HNW), jnp.float32),
    )(h_neigh, W_preagg)

    EB = 1024
    ew = pl.pallas_call(
        _edge_body,
        grid=(E_PAD // EB,),
        in_specs=[pl.BlockSpec((EB, DE), lambda i: (jnp.minimum(i, (E - 1) // EB), 0)),
                  pl.BlockSpec((DOUT * DOUT, DE), lambda i: (0, 0)),
                  pl.BlockSpec((1, DOUT * DOUT), lambda i: (0, 0))],
        out_specs=pl.BlockSpec((EB, DOUT), lambda i: (i, 0)),
        out_shape=jax.ShapeDtypeStruct((E_PAD, DOUT), jnp.float32),
    )(edge_features, W_edge.astype(jnp.bfloat16),
      b_edge.reshape(1, DOUT * DOUT))

    acc = _sc_edge_scatter(hn, src_pad, dst_pad, ew)

    z = pl.pallas_call(
        _final_body,
        grid=(10,),
        in_specs=[pl.BlockSpec((1000, DIN), lambda i: (i, 0)),
                  pl.BlockSpec((NC, 1000, AW), lambda i: (0, i, 0)),
                  pl.BlockSpec((DOUT, DIN), lambda i: (0, 0)),
                  pl.BlockSpec((DOUT, DOUT), lambda i: (0, 0)),
                  pl.BlockSpec((DOUT, DOUT), lambda i: (0, 0))],
        out_specs=pl.BlockSpec((1000, DOUT), lambda i: (i, 0)),
        out_shape=jax.ShapeDtypeStruct((N, DOUT), jnp.float32),
    )(h_self, acc, W_preagg, W_self, W_neigh)
    return z


# speedup vs baseline: 1.0859x; 1.0859x over previous
import functools

import jax
import jax.numpy as jnp
from jax import lax
from jax.experimental import pallas as pl
from jax.experimental.pallas import tpu as pltpu
from jax.experimental.pallas import tpu_sc as plsc

N = 10000
E = 50000
DIN = 256
DOUT = 32
DE = 16
HNW = DOUT

NC = 2
NS = 16
NW = NC * NS
CHUNK = 128
QPC = CHUNK // 4
CH_PER_TILE = 13
E_TILE = CHUNK * CH_PER_TILE
E_PAD = NW * E_TILE
N_PAD = 10240
ROWS_TILE = N_PAD // NS
AW = 48

_SC_MESH = plsc.VectorSubcoreMesh(
    core_axis_name="c", subcore_axis_name="s", num_cores=NC, num_subcores=NS)


@functools.partial(
    pl.kernel,
    out_type=jax.ShapeDtypeStruct((NC, N_PAD, AW), jnp.float32),
    mesh=_SC_MESH,
    compiler_params=pltpu.CompilerParams(use_tc_tiling_on_sc=False),
    scratch_types=[
        pltpu.VMEM((CH_PER_TILE, 1, CHUNK), jnp.int32),
        pltpu.VMEM((CH_PER_TILE, 1, CHUNK), jnp.int32),
        pltpu.VMEM((E_TILE, DOUT), jnp.float32),
        pltpu.VMEM((CHUNK, HNW), jnp.float32),
        pltpu.VMEM((CHUNK, HNW), jnp.float32),
        pltpu.VMEM((CHUNK, AW), jnp.float32),
        pltpu.VMEM((CHUNK, AW), jnp.float32),
        pltpu.VMEM_SHARED((N_PAD, AW), jnp.float32),
        pltpu.SemaphoreType.DMA,
        pltpu.SemaphoreType.DMA,
        pltpu.SemaphoreType.DMA,
        pltpu.SemaphoreType.DMA,
        pltpu.SemaphoreType.DMA,
    ],
)
def _sc_edge_scatter(hn_hbm, src_hbm, dst_hbm, ew_hbm, out_hbm,
                     idx_src, idx_dst, ew_all, rows0, rows1, msg0, msg1,
                     acc_sp, esem, gsem0, gsem1, ssem0, ssem1):
    c = lax.axis_index("c")
    s = lax.axis_index("s")
    wid = c * NS + s

    rows = (rows0, rows1)
    msg = (msg0, msg1)
    gsem = (gsem0, gsem1)
    ssem = (ssem0, ssem1)

    ew_cp = pltpu.async_copy(
        ew_hbm.at[pl.ds(wid * E_TILE, E_TILE)], ew_all, esem)
    pltpu.sync_copy(src_hbm.at[pl.ds(wid * CH_PER_TILE, CH_PER_TILE)], idx_src)
    pltpu.sync_copy(dst_hbm.at[pl.ds(wid * CH_PER_TILE, CH_PER_TILE)], idx_dst)

    zeros16 = jnp.zeros((16,), jnp.float32)

    def _zero_row(i, _):
        msg0[i, pl.ds(0, 16)] = zeros16
        msg0[i, pl.ds(16, 16)] = zeros16
        msg0[i, pl.ds(32, 16)] = zeros16
        return 0
    lax.fori_loop(0, CHUNK, _zero_row, 0)

    zcps = [
        pltpu.async_copy(
            msg0, acc_sp.at[pl.ds(s * ROWS_TILE + j * CHUNK, CHUNK)],
            (gsem0, gsem1)[j & 1])
        for j in range(ROWS_TILE // CHUNK)
    ]
    for zc in zcps:
        zc.wait()

    deg_lane = jnp.where(lax.iota(jnp.int32, 16) == 0,
                         jnp.float32(1.0), jnp.float32(0.0))

    def _set_deg(i, _):
        msg0[i, pl.ds(DOUT, 16)] = deg_lane
        msg1[i, pl.ds(DOUT, 16)] = deg_lane
        return 0
    lax.fori_loop(0, CHUNK, _set_deg, 0)

    ew_cp.wait()
    plsc.subcore_barrier()

    gcp = [None, None]
    scp = [None, None]
    gcp[0] = pltpu.async_copy(hn_hbm.at[idx_src.at[0, 0]], rows0, gsem0)
    for j in range(CH_PER_TILE):
        b = j & 1
        nb = b ^ 1
        if j + 1 < CH_PER_TILE:
            gcp[nb] = pltpu.async_copy(
                hn_hbm.at[idx_src.at[j + 1, 0]], rows[nb], gsem[nb])
        gcp[b].wait()
        if scp[b] is not None:
            scp[b].wait()
        rb = rows[b]
        mb = msg[b]

        e0 = j * CHUNK

        def _mul(q, _):
            r = q * 4
            for k in range(4):
                for h in range(2):
                    mb[r + k, pl.ds(h * 16, 16)] = (
                        rb[r + k, pl.ds(h * 16, 16)]
                        * ew_all[e0 + r + k, pl.ds(h * 16, 16)])
            return 0
        lax.fori_loop(0, QPC, _mul, 0)
        scp[b] = pltpu.async_copy(mb, acc_sp.at[idx_dst.at[j, 0]], ssem[b], add=True)
    scp[0].wait()
    scp[1].wait()
    plsc.subcore_barrier()

    wcp = [None, None]
    for j in range(ROWS_TILE // CHUNK):
        b = j & 1
        r0 = s * ROWS_TILE + j * CHUNK
        if wcp[b] is not None:
            wcp[b].wait()
        pltpu.async_copy(acc_sp.at[pl.ds(r0, CHUNK)], msg[b], gsem[b]).wait()
        wcp[b] = pltpu.async_copy(msg[b], out_hbm.at[c, pl.ds(r0, CHUNK)], ssem[b])
    for w in wcp:
        if w is not None:
            w.wait()


def _preagg_body(x_ref, w_ref, o_ref):
    y = lax.dot_general(x_ref[...], w_ref[...], (((1,), (1,)), ((), ())),
                        preferred_element_type=jnp.float32)
    o_ref[...] = jnp.maximum(y, 0.0)


def _edge_body(ef_ref, w_ref, b_ref, o_ref):
    y = lax.dot_general(ef_ref[...].astype(jnp.bfloat16), w_ref[...],
                        (((1,), (1,)), ((), ())),
                        preferred_element_type=jnp.float32)
    y = jnp.maximum(y + b_ref[...], 0.0)
    for half in (512, 256, 128, 64, 32):
        y = y[:, :half] + y[:, half:2 * half]
    o_ref[...] = y


def _final_body(hs_ref, acc_ref, wp_ref, ws_ref, wn_ref, o_ref):
    hs = jnp.maximum(
        lax.dot_general(hs_ref[...], wp_ref[...], (((1,), (1,)), ((), ())),
                        preferred_element_type=jnp.float32), 0.0)
    a = acc_ref[0] + acc_ref[1]
    neigh = a[:, :DOUT] / jnp.maximum(a[:, DOUT:DOUT + 1], 1.0)
    z1 = jnp.maximum(
        lax.dot_general(hs, ws_ref[...], (((1,), (1,)), ((), ())),
                        preferred_element_type=jnp.float32), 0.0)
    z2 = jnp.maximum(
        lax.dot_general(neigh, wn_ref[...], (((1,), (1,)), ((), ())),
                        preferred_element_type=jnp.float32), 0.0)
    o_ref[...] = jnp.maximum(z1 + z2, 0.0)


def kernel(h_neigh, h_self, edge_features, W_preagg, W_self, W_neigh,
           W_edge, b_edge, edge_index):
    src = edge_index[0]
    dst = edge_index[1]
    npad = E_PAD - E
    pad_dst = N + (jnp.arange(npad, dtype=jnp.int32) % (N_PAD - N))
    pad_src = jnp.arange(npad, dtype=jnp.int32) % N
    src_pad = jnp.concatenate(
        [src, pad_src]).reshape(NW * CH_PER_TILE, 1, CHUNK)
    dst_pad = jnp.concatenate(
        [dst, pad_dst]).reshape(NW * CH_PER_TILE, 1, CHUNK)
    hn = pl.pallas_call(
        _preagg_body,
        grid=(5,),
        in_specs=[pl.BlockSpec((2000, DIN), lambda i: (i, 0)),
                  pl.BlockSpec((HNW, DIN), lambda i: (0, 0))],
        out_specs=pl.BlockSpec((2000, HNW), lambda i: (i, 0)),
        out_shape=jax.ShapeDtypeStruct((N, HNW), jnp.float32),
    )(h_neigh, W_preagg)

    EB = 2048
    ew = pl.pallas_call(
        _edge_body,
        grid=(E_PAD // EB,),
        in_specs=[pl.BlockSpec((EB, DE), lambda i: (jnp.minimum(i, (E - 1) // EB), 0)),
                  pl.BlockSpec((DOUT * DOUT, DE), lambda i: (0, 0)),
                  pl.BlockSpec((1, DOUT * DOUT), lambda i: (0, 0))],
        out_specs=pl.BlockSpec((EB, DOUT), lambda i: (i, 0)),
        out_shape=jax.ShapeDtypeStruct((E_PAD, DOUT), jnp.float32),
    )(edge_features, W_edge.astype(jnp.bfloat16),
      b_edge.reshape(1, DOUT * DOUT))

    acc = _sc_edge_scatter(hn, src_pad, dst_pad, ew)

    z = pl.pallas_call(
        _final_body,
        grid=(10,),
        in_specs=[pl.BlockSpec((1000, DIN), lambda i: (i, 0)),
                  pl.BlockSpec((NC, 1000, AW), lambda i: (0, i, 0)),
                  pl.BlockSpec((DOUT, DIN), lambda i: (0, 0)),
                  pl.BlockSpec((DOUT, DOUT), lambda i: (0, 0)),
                  pl.BlockSpec((DOUT, DOUT), lambda i: (0, 0))],
        out_specs=pl.BlockSpec((1000, DOUT), lambda i: (i, 0)),
        out_shape=jax.ShapeDtypeStruct((N, DOUT), jnp.float32),
    )(h_self, acc, W_preagg, W_self, W_neigh)
    return z


# speedup vs baseline: 1.1818x; 1.0883x over previous
import functools

import jax
import jax.numpy as jnp
from jax import lax
from jax.experimental import pallas as pl
from jax.experimental.pallas import tpu as pltpu
from jax.experimental.pallas import tpu_sc as plsc

N = 10000
E = 50000
DIN = 256
DOUT = 32
DE = 16
HNW = DOUT

NC = 2
NS = 16
NW = NC * NS
CHUNK = 128
QPC = CHUNK // 4
CH_PER_TILE = 13
E_TILE = CHUNK * CH_PER_TILE
E_PAD = NW * E_TILE
N_PAD = 10240
ROWS_TILE = N_PAD // NS
AW = 48

_SC_MESH = plsc.VectorSubcoreMesh(
    core_axis_name="c", subcore_axis_name="s", num_cores=NC, num_subcores=NS)


@functools.partial(
    pl.kernel,
    out_type=jax.ShapeDtypeStruct((NC, N_PAD, AW), jnp.float32),
    mesh=_SC_MESH,
    compiler_params=pltpu.CompilerParams(use_tc_tiling_on_sc=False),
    scratch_types=[
        pltpu.VMEM((CH_PER_TILE, 1, CHUNK), jnp.int32),
        pltpu.VMEM((CH_PER_TILE, 1, CHUNK), jnp.int32),
        pltpu.VMEM((E_TILE, DOUT), jnp.float32),
        pltpu.VMEM((CHUNK, HNW), jnp.float32),
        pltpu.VMEM((CHUNK, HNW), jnp.float32),
        pltpu.VMEM((CHUNK, AW), jnp.float32),
        pltpu.VMEM((CHUNK, AW), jnp.float32),
        pltpu.VMEM_SHARED((N_PAD, AW), jnp.float32),
        pltpu.SemaphoreType.DMA,
        pltpu.SemaphoreType.DMA,
        pltpu.SemaphoreType.DMA,
        pltpu.SemaphoreType.DMA,
        pltpu.SemaphoreType.DMA,
    ],
)
def _sc_edge_scatter(hn_hbm, src_hbm, dst_hbm, ew_hbm, out_hbm,
                     idx_src, idx_dst, ew_all, rows0, rows1, msg0, msg1,
                     acc_sp, esem, gsem0, gsem1, ssem0, ssem1):
    c = lax.axis_index("c")
    s = lax.axis_index("s")
    wid = c * NS + s

    rows = (rows0, rows1)
    msg = (msg0, msg1)
    gsem = (gsem0, gsem1)
    ssem = (ssem0, ssem1)

    ew_cp = pltpu.async_copy(
        ew_hbm.at[pl.ds(wid * E_TILE, E_TILE)], ew_all, esem)
    pltpu.sync_copy(src_hbm.at[pl.ds(wid * CH_PER_TILE, CH_PER_TILE)], idx_src)
    pltpu.sync_copy(dst_hbm.at[pl.ds(wid * CH_PER_TILE, CH_PER_TILE)], idx_dst)

    zeros16 = jnp.zeros((16,), jnp.float32)

    def _zero_row(i, _):
        msg0[i, pl.ds(0, 16)] = zeros16
        msg0[i, pl.ds(16, 16)] = zeros16
        msg0[i, pl.ds(32, 16)] = zeros16
        return 0
    lax.fori_loop(0, CHUNK, _zero_row, 0)

    zcps = [
        pltpu.async_copy(
            msg0, acc_sp.at[pl.ds(s * ROWS_TILE + j * CHUNK, CHUNK)],
            (gsem0, gsem1)[j & 1])
        for j in range(ROWS_TILE // CHUNK)
    ]
    for zc in zcps:
        zc.wait()

    deg_lane = jnp.where(lax.iota(jnp.int32, 16) == 0,
                         jnp.float32(1.0), jnp.float32(0.0))

    def _set_deg(i, _):
        msg0[i, pl.ds(DOUT, 16)] = deg_lane
        msg1[i, pl.ds(DOUT, 16)] = deg_lane
        return 0
    lax.fori_loop(0, CHUNK, _set_deg, 0)

    ew_cp.wait()
    plsc.subcore_barrier()

    gcp = [None, None]
    scp = [None, None]
    gcp[0] = pltpu.async_copy(hn_hbm.at[idx_src.at[0, 0]], rows0, gsem0)
    for j in range(CH_PER_TILE):
        b = j & 1
        nb = b ^ 1
        if j + 1 < CH_PER_TILE:
            gcp[nb] = pltpu.async_copy(
                hn_hbm.at[idx_src.at[j + 1, 0]], rows[nb], gsem[nb])
        gcp[b].wait()
        if scp[b] is not None:
            scp[b].wait()
        rb = rows[b]
        mb = msg[b]

        e0 = j * CHUNK

        def _mul(q, _):
            r = q * 4
            for k in range(4):
                for h in range(2):
                    mb[r + k, pl.ds(h * 16, 16)] = (
                        rb[r + k, pl.ds(h * 16, 16)]
                        * ew_all[e0 + r + k, pl.ds(h * 16, 16)])
            return 0
        lax.fori_loop(0, QPC, _mul, 0)
        scp[b] = pltpu.async_copy(mb, acc_sp.at[idx_dst.at[j, 0]], ssem[b], add=True)
    scp[0].wait()
    scp[1].wait()
    plsc.subcore_barrier()

    wcp = [None, None]
    for j in range(ROWS_TILE // CHUNK):
        b = j & 1
        r0 = s * ROWS_TILE + j * CHUNK
        if wcp[b] is not None:
            wcp[b].wait()
        pltpu.async_copy(acc_sp.at[pl.ds(r0, CHUNK)], msg[b], gsem[b]).wait()
        wcp[b] = pltpu.async_copy(msg[b], out_hbm.at[c, pl.ds(r0, CHUNK)], ssem[b])
    for w in wcp:
        if w is not None:
            w.wait()


def _preagg_body(x_ref, w_ref, o_ref):
    y = lax.dot_general(x_ref[...], w_ref[...], (((1,), (1,)), ((), ())),
                        preferred_element_type=jnp.float32)
    o_ref[...] = jnp.maximum(y, 0.0)


def _edge_body(ef_ref, w_ref, b_ref, o_ref):
    y = lax.dot_general(ef_ref[...].astype(jnp.bfloat16), w_ref[...],
                        (((0,), (1,)), ((), ())),
                        preferred_element_type=jnp.float32)
    y = jnp.maximum(y + b_ref[...], 0.0)
    for half in (512, 256, 128, 64, 32):
        y = y[:, :half] + y[:, half:2 * half]
    o_ref[...] = y


def _final_body(hs_ref, acc_ref, wp_ref, ws_ref, wn_ref, o_ref):
    hs = jnp.maximum(
        lax.dot_general(hs_ref[...], wp_ref[...], (((1,), (1,)), ((), ())),
                        preferred_element_type=jnp.float32), 0.0)
    a = acc_ref[0] + acc_ref[1]
    neigh = a[:, :DOUT] / jnp.maximum(a[:, DOUT:DOUT + 1], 1.0)
    z1 = jnp.maximum(
        lax.dot_general(hs, ws_ref[...], (((1,), (1,)), ((), ())),
                        preferred_element_type=jnp.float32), 0.0)
    z2 = jnp.maximum(
        lax.dot_general(neigh, wn_ref[...], (((1,), (1,)), ((), ())),
                        preferred_element_type=jnp.float32), 0.0)
    o_ref[...] = jnp.maximum(z1 + z2, 0.0)


def kernel(h_neigh, h_self, edge_features, W_preagg, W_self, W_neigh,
           W_edge, b_edge, edge_index):
    src = edge_index[0]
    dst = edge_index[1]
    npad = E_PAD - E
    pad_dst = N + (jnp.arange(npad, dtype=jnp.int32) % (N_PAD - N))
    pad_src = jnp.arange(npad, dtype=jnp.int32) % N
    src_pad = jnp.concatenate(
        [src, pad_src]).reshape(NW * CH_PER_TILE, 1, CHUNK)
    dst_pad = jnp.concatenate(
        [dst, pad_dst]).reshape(NW * CH_PER_TILE, 1, CHUNK)
    hn = pl.pallas_call(
        _preagg_body,
        grid=(5,),
        in_specs=[pl.BlockSpec((2000, DIN), lambda i: (i, 0)),
                  pl.BlockSpec((HNW, DIN), lambda i: (0, 0))],
        out_specs=pl.BlockSpec((2000, HNW), lambda i: (i, 0)),
        out_shape=jax.ShapeDtypeStruct((N, HNW), jnp.float32),
    )(h_neigh, W_preagg)

    EB = 2048
    ew = pl.pallas_call(
        _edge_body,
        grid=(E_PAD // EB,),
        in_specs=[pl.BlockSpec((DE, EB), lambda i: (0, jnp.minimum(i, (E - 1) // EB))),
                  pl.BlockSpec((DOUT * DOUT, DE), lambda i: (0, 0)),
                  pl.BlockSpec((1, DOUT * DOUT), lambda i: (0, 0))],
        out_specs=pl.BlockSpec((EB, DOUT), lambda i: (i, 0)),
        out_shape=jax.ShapeDtypeStruct((E_PAD, DOUT), jnp.float32),
    )(edge_features.T, W_edge.astype(jnp.bfloat16),
      b_edge.reshape(1, DOUT * DOUT))

    acc = _sc_edge_scatter(hn, src_pad, dst_pad, ew)

    z = pl.pallas_call(
        _final_body,
        grid=(10,),
        in_specs=[pl.BlockSpec((1000, DIN), lambda i: (i, 0)),
                  pl.BlockSpec((NC, 1000, AW), lambda i: (0, i, 0)),
                  pl.BlockSpec((DOUT, DIN), lambda i: (0, 0)),
                  pl.BlockSpec((DOUT, DOUT), lambda i: (0, 0)),
                  pl.BlockSpec((DOUT, DOUT), lambda i: (0, 0))],
        out_specs=pl.BlockSpec((1000, DOUT), lambda i: (i, 0)),
        out_shape=jax.ShapeDtypeStruct((N, DOUT), jnp.float32),
    )(h_self, acc, W_preagg, W_self, W_neigh)
    return z


# speedup vs baseline: 1.3318x; 1.1269x over previous
import functools

import jax
import jax.numpy as jnp
from jax import lax
from jax.experimental import pallas as pl
from jax.experimental.pallas import tpu as pltpu
from jax.experimental.pallas import tpu_sc as plsc

N = 10000
E = 50000
DIN = 256
DOUT = 32
DE = 16
HNW = DOUT

NC = 2
NS = 16
NW = NC * NS
CHUNK = 128
QPC = CHUNK // 4
CH_PER_TILE = 13
E_TILE = CHUNK * CH_PER_TILE
E_PAD = NW * E_TILE
N_PAD = 10240
ROWS_TILE = N_PAD // NS
AW = 48

_SC_MESH = plsc.VectorSubcoreMesh(
    core_axis_name="c", subcore_axis_name="s", num_cores=NC, num_subcores=NS)


@functools.partial(
    pl.kernel,
    out_type=jax.ShapeDtypeStruct((NC, N_PAD, AW), jnp.float32),
    mesh=_SC_MESH,
    compiler_params=pltpu.CompilerParams(use_tc_tiling_on_sc=False),
    scratch_types=[
        pltpu.VMEM((CH_PER_TILE, 1, CHUNK), jnp.int32),
        pltpu.VMEM((CH_PER_TILE, 1, CHUNK), jnp.int32),
        pltpu.VMEM((E_TILE, DOUT), jnp.float32),
        pltpu.VMEM((CHUNK, HNW), jnp.float32),
        pltpu.VMEM((CHUNK, HNW), jnp.float32),
        pltpu.VMEM((CHUNK, AW), jnp.float32),
        pltpu.VMEM((CHUNK, AW), jnp.float32),
        pltpu.VMEM_SHARED((N_PAD, AW), jnp.float32),
        pltpu.SemaphoreType.DMA,
        pltpu.SemaphoreType.DMA,
        pltpu.SemaphoreType.DMA,
        pltpu.SemaphoreType.DMA,
        pltpu.SemaphoreType.DMA,
    ],
)
def _sc_edge_scatter(hn_hbm, src_hbm, dst_hbm, ew_hbm, out_hbm,
                     idx_src, idx_dst, ew_all, rows0, rows1, msg0, msg1,
                     acc_sp, esem, gsem0, gsem1, ssem0, ssem1):
    c = lax.axis_index("c")
    s = lax.axis_index("s")
    wid = c * NS + s

    rows = (rows0, rows1)
    msg = (msg0, msg1)
    gsem = (gsem0, gsem1)
    ssem = (ssem0, ssem1)

    g = wid // 8
    ew_cp = pltpu.async_copy(
        ew_hbm.at[pl.ds((wid % 8) * E_TILE, E_TILE), pl.ds(g * DOUT, DOUT)],
        ew_all, esem)
    pltpu.sync_copy(src_hbm.at[pl.ds(wid * CH_PER_TILE, CH_PER_TILE)], idx_src)
    pltpu.sync_copy(dst_hbm.at[pl.ds(wid * CH_PER_TILE, CH_PER_TILE)], idx_dst)

    zeros16 = jnp.zeros((16,), jnp.float32)

    def _zero_row(i, _):
        msg0[i, pl.ds(0, 16)] = zeros16
        msg0[i, pl.ds(16, 16)] = zeros16
        msg0[i, pl.ds(32, 16)] = zeros16
        return 0
    lax.fori_loop(0, CHUNK, _zero_row, 0)

    zcps = [
        pltpu.async_copy(
            msg0, acc_sp.at[pl.ds(s * ROWS_TILE + j * CHUNK, CHUNK)],
            (gsem0, gsem1)[j & 1])
        for j in range(ROWS_TILE // CHUNK)
    ]
    for zc in zcps:
        zc.wait()

    deg_lane = jnp.where(lax.iota(jnp.int32, 16) == 0,
                         jnp.float32(1.0), jnp.float32(0.0))

    def _set_deg(i, _):
        msg0[i, pl.ds(DOUT, 16)] = deg_lane
        msg1[i, pl.ds(DOUT, 16)] = deg_lane
        return 0
    lax.fori_loop(0, CHUNK, _set_deg, 0)

    ew_cp.wait()
    plsc.subcore_barrier()

    gcp = [None, None]
    scp = [None, None]
    gcp[0] = pltpu.async_copy(hn_hbm.at[idx_src.at[0, 0]], rows0, gsem0)
    for j in range(CH_PER_TILE):
        b = j & 1
        nb = b ^ 1
        if j + 1 < CH_PER_TILE:
            gcp[nb] = pltpu.async_copy(
                hn_hbm.at[idx_src.at[j + 1, 0]], rows[nb], gsem[nb])
        gcp[b].wait()
        if scp[b] is not None:
            scp[b].wait()
        rb = rows[b]
        mb = msg[b]

        e0 = j * CHUNK

        def _mul(q, _):
            r = q * 4
            for k in range(4):
                for h in range(2):
                    mb[r + k, pl.ds(h * 16, 16)] = (
                        rb[r + k, pl.ds(h * 16, 16)]
                        * ew_all[e0 + r + k, pl.ds(h * 16, 16)])
            return 0
        lax.fori_loop(0, QPC, _mul, 0)
        scp[b] = pltpu.async_copy(mb, acc_sp.at[idx_dst.at[j, 0]], ssem[b], add=True)
    scp[0].wait()
    scp[1].wait()
    plsc.subcore_barrier()

    wcp = [None, None]
    for j in range(ROWS_TILE // CHUNK):
        b = j & 1
        r0 = s * ROWS_TILE + j * CHUNK
        if wcp[b] is not None:
            wcp[b].wait()
        pltpu.async_copy(acc_sp.at[pl.ds(r0, CHUNK)], msg[b], gsem[b]).wait()
        wcp[b] = pltpu.async_copy(msg[b], out_hbm.at[c, pl.ds(r0, CHUNK)], ssem[b])
    for w in wcp:
        if w is not None:
            w.wait()


def _preagg_body(x_ref, w_ref, o_ref):
    y = lax.dot_general(x_ref[...], w_ref[...], (((1,), (1,)), ((), ())),
                        preferred_element_type=jnp.float32)
    o_ref[...] = jnp.maximum(y, 0.0)


def _edge_body(ef0, ef1, ef2, ef3, w_ref, b_ref, o_ref):
    ys = []
    for ef_ref in (ef0, ef1, ef2, ef3):
        y = lax.dot_general(ef_ref[...].astype(jnp.bfloat16), w_ref[...],
                            (((0,), (1,)), ((), ())),
                            preferred_element_type=jnp.float32)
        y = jnp.maximum(y + b_ref[...], 0.0)
        for half in (512, 256, 128, 64, 32):
            y = y[:, :half] + y[:, half:2 * half]
        ys.append(y)
    o_ref[...] = jnp.concatenate(ys, axis=1)


def _final_body(hs_ref, acc_ref, wp_ref, ws_ref, wn_ref, o_ref):
    hs = jnp.maximum(
        lax.dot_general(hs_ref[...], wp_ref[...], (((1,), (1,)), ((), ())),
                        preferred_element_type=jnp.float32), 0.0)
    a = acc_ref[0] + acc_ref[1]
    neigh = a[:, :DOUT] / jnp.maximum(a[:, DOUT:DOUT + 1], 1.0)
    z1 = jnp.maximum(
        lax.dot_general(hs, ws_ref[...], (((1,), (1,)), ((), ())),
                        preferred_element_type=jnp.float32), 0.0)
    z2 = jnp.maximum(
        lax.dot_general(neigh, wn_ref[...], (((1,), (1,)), ((), ())),
                        preferred_element_type=jnp.float32), 0.0)
    o_ref[...] = jnp.maximum(z1 + z2, 0.0)


def kernel(h_neigh, h_self, edge_features, W_preagg, W_self, W_neigh,
           W_edge, b_edge, edge_index):
    src = edge_index[0]
    dst = edge_index[1]
    npad = E_PAD - E
    pad_dst = N + (jnp.arange(npad, dtype=jnp.int32) % (N_PAD - N))
    pad_src = jnp.arange(npad, dtype=jnp.int32) % N
    src_pad = jnp.concatenate(
        [src, pad_src]).reshape(NW * CH_PER_TILE, 1, CHUNK)
    dst_pad = jnp.concatenate(
        [dst, pad_dst]).reshape(NW * CH_PER_TILE, 1, CHUNK)
    hn = pl.pallas_call(
        _preagg_body,
        grid=(5,),
        in_specs=[pl.BlockSpec((2000, DIN), lambda i: (i, 0)),
                  pl.BlockSpec((HNW, DIN), lambda i: (0, 0))],
        out_specs=pl.BlockSpec((2000, HNW), lambda i: (i, 0)),
        out_shape=jax.ShapeDtypeStruct((N, HNW), jnp.float32),
    )(h_neigh, W_preagg)

    EB4 = 512
    NB4 = (E_PAD // 4) // EB4
    LASTB = (E - 1) // EB4

    def _mk(k):
        return lambda i: (0, jnp.minimum(i + NB4 * k, LASTB))

    ew = pl.pallas_call(
        _edge_body,
        grid=(NB4,),
        in_specs=[pl.BlockSpec((DE, EB4), _mk(0)),
                  pl.BlockSpec((DE, EB4), _mk(1)),
                  pl.BlockSpec((DE, EB4), _mk(2)),
                  pl.BlockSpec((DE, EB4), _mk(3)),
                  pl.BlockSpec((DOUT * DOUT, DE), lambda i: (0, 0)),
                  pl.BlockSpec((1, DOUT * DOUT), lambda i: (0, 0))],
        out_specs=pl.BlockSpec((EB4, 128), lambda i: (i, 0)),
        out_shape=jax.ShapeDtypeStruct((E_PAD // 4, 128), jnp.float32),
    )(edge_features.T, edge_features.T, edge_features.T, edge_features.T,
      W_edge.astype(jnp.bfloat16), b_edge.reshape(1, DOUT * DOUT))

    acc = _sc_edge_scatter(hn, src_pad, dst_pad, ew)

    z = pl.pallas_call(
        _final_body,
        grid=(10,),
        in_specs=[pl.BlockSpec((1000, DIN), lambda i: (i, 0)),
                  pl.BlockSpec((NC, 1000, AW), lambda i: (0, i, 0)),
                  pl.BlockSpec((DOUT, DIN), lambda i: (0, 0)),
                  pl.BlockSpec((DOUT, DOUT), lambda i: (0, 0)),
                  pl.BlockSpec((DOUT, DOUT), lambda i: (0, 0))],
        out_specs=pl.BlockSpec((1000, DOUT), lambda i: (i, 0)),
        out_shape=jax.ShapeDtypeStruct((N, DOUT), jnp.float32),
    )(h_self, acc, W_preagg, W_self, W_neigh)
    return z


# speedup vs baseline: 1.4258x; 1.0706x over previous
import functools

import jax
import jax.numpy as jnp
from jax import lax
from jax.experimental import pallas as pl
from jax.experimental.pallas import tpu as pltpu
from jax.experimental.pallas import tpu_sc as plsc

N = 10000
E = 50000
DIN = 256
DOUT = 32
DE = 16
HNW = DOUT

NC = 2
NS = 16
NW = NC * NS
CHUNK = 128
QPC = CHUNK // 4
CH_PER_TILE = 13
E_TILE = CHUNK * CH_PER_TILE
E_PAD = NW * E_TILE
N_PAD = 10240
ROWS_TILE = N_PAD // NS
AW = 48

_SC_MESH = plsc.VectorSubcoreMesh(
    core_axis_name="c", subcore_axis_name="s", num_cores=NC, num_subcores=NS)


@functools.partial(
    pl.kernel,
    out_type=jax.ShapeDtypeStruct((NC, N_PAD, AW), jnp.float32),
    mesh=_SC_MESH,
    compiler_params=pltpu.CompilerParams(use_tc_tiling_on_sc=False),
    scratch_types=[
        pltpu.VMEM((CH_PER_TILE, 1, CHUNK), jnp.int32),
        pltpu.VMEM((CH_PER_TILE, 1, CHUNK), jnp.int32),
        pltpu.VMEM((E_TILE, DOUT), jnp.float32),
        pltpu.VMEM((CHUNK, HNW), jnp.float32),
        pltpu.VMEM((CHUNK, HNW), jnp.float32),
        pltpu.VMEM((CHUNK, AW), jnp.float32),
        pltpu.VMEM((CHUNK, AW), jnp.float32),
        pltpu.VMEM_SHARED((N_PAD, AW), jnp.float32),
        pltpu.SemaphoreType.DMA,
        pltpu.SemaphoreType.DMA,
        pltpu.SemaphoreType.DMA,
        pltpu.SemaphoreType.DMA,
        pltpu.SemaphoreType.DMA,
    ],
)
def _sc_edge_scatter(hn_hbm, src_hbm, dst_hbm, ew_hbm, out_hbm,
                     idx_src, idx_dst, ew_all, rows0, rows1, msg0, msg1,
                     acc_sp, esem, gsem0, gsem1, ssem0, ssem1):
    c = lax.axis_index("c")
    s = lax.axis_index("s")
    wid = c * NS + s

    rows = (rows0, rows1)
    msg = (msg0, msg1)
    gsem = (gsem0, gsem1)
    ssem = (ssem0, ssem1)

    g = wid // 8
    ew_cp = pltpu.async_copy(
        ew_hbm.at[pl.ds((wid % 8) * E_TILE, E_TILE), pl.ds(g * DOUT, DOUT)],
        ew_all, esem)
    pltpu.sync_copy(src_hbm.at[pl.ds(wid * CH_PER_TILE, CH_PER_TILE)], idx_src)
    pltpu.sync_copy(dst_hbm.at[pl.ds(wid * CH_PER_TILE, CH_PER_TILE)], idx_dst)

    zeros16 = jnp.zeros((16,), jnp.float32)

    def _zero_row(i, _):
        msg0[i, pl.ds(0, 16)] = zeros16
        msg0[i, pl.ds(16, 16)] = zeros16
        msg0[i, pl.ds(32, 16)] = zeros16
        return 0
    lax.fori_loop(0, CHUNK, _zero_row, 0)

    zcps = [
        pltpu.async_copy(
            msg0, acc_sp.at[pl.ds(s * ROWS_TILE + j * CHUNK, CHUNK)],
            (gsem0, gsem1)[j & 1])
        for j in range(ROWS_TILE // CHUNK)
    ]
    for zc in zcps:
        zc.wait()

    deg_lane = jnp.where(lax.iota(jnp.int32, 16) == 0,
                         jnp.float32(1.0), jnp.float32(0.0))

    def _set_deg(i, _):
        msg0[i, pl.ds(DOUT, 16)] = deg_lane
        msg1[i, pl.ds(DOUT, 16)] = deg_lane
        return 0
    lax.fori_loop(0, CHUNK, _set_deg, 0)

    ew_cp.wait()
    plsc.subcore_barrier()

    gcp = [None, None]
    scp = [None, None]
    gcp[0] = pltpu.async_copy(hn_hbm.at[idx_src.at[0, 0]], rows0, gsem0)
    for j in range(CH_PER_TILE):
        b = j & 1
        nb = b ^ 1
        if j + 1 < CH_PER_TILE:
            gcp[nb] = pltpu.async_copy(
                hn_hbm.at[idx_src.at[j + 1, 0]], rows[nb], gsem[nb])
        gcp[b].wait()
        if scp[b] is not None:
            scp[b].wait()
        rb = rows[b]
        mb = msg[b]

        e0 = j * CHUNK

        @plsc.parallel_loop(0, QPC, unroll=2)
        def _mul(q):
            r = q * 4
            for k in range(4):
                for h in range(2):
                    mb[r + k, pl.ds(h * 16, 16)] = (
                        rb[r + k, pl.ds(h * 16, 16)]
                        * ew_all[e0 + r + k, pl.ds(h * 16, 16)])
        scp[b] = pltpu.async_copy(mb, acc_sp.at[idx_dst.at[j, 0]], ssem[b], add=True)
    scp[0].wait()
    scp[1].wait()
    plsc.subcore_barrier()

    wcp = [None, None]
    for j in range(ROWS_TILE // CHUNK):
        b = j & 1
        r0 = s * ROWS_TILE + j * CHUNK
        if wcp[b] is not None:
            wcp[b].wait()
        pltpu.async_copy(acc_sp.at[pl.ds(r0, CHUNK)], msg[b], gsem[b]).wait()
        wcp[b] = pltpu.async_copy(msg[b], out_hbm.at[c, pl.ds(r0, CHUNK)], ssem[b])
    for w in wcp:
        if w is not None:
            w.wait()


def _preagg_body(x_ref, w_ref, o_ref):
    y = lax.dot_general(x_ref[...], w_ref[...], (((1,), (1,)), ((), ())),
                        preferred_element_type=jnp.float32)
    o_ref[...] = jnp.maximum(y, 0.0)


def _edge_body(ef0, ef1, ef2, ef3, w_ref, b_ref, o_ref):
    ys = []
    for ef_ref in (ef0, ef1, ef2, ef3):
        y = lax.dot_general(ef_ref[...].astype(jnp.bfloat16), w_ref[...],
                            (((0,), (1,)), ((), ())),
                            preferred_element_type=jnp.float32)
        y = jnp.maximum(y + b_ref[...], 0.0)
        for half in (512, 256, 128, 64, 32):
            y = y[:, :half] + y[:, half:2 * half]
        ys.append(y)
    o_ref[...] = jnp.concatenate(ys, axis=1)


def _final_body(hs_ref, acc_ref, wp_ref, ws_ref, wn_ref, o_ref):
    hs = jnp.maximum(
        lax.dot_general(hs_ref[...], wp_ref[...], (((1,), (1,)), ((), ())),
                        preferred_element_type=jnp.float32), 0.0)
    a = acc_ref[0] + acc_ref[1]
    neigh = a[:, :DOUT] / jnp.maximum(a[:, DOUT:DOUT + 1], 1.0)
    z1 = jnp.maximum(
        lax.dot_general(hs, ws_ref[...], (((1,), (1,)), ((), ())),
                        preferred_element_type=jnp.float32), 0.0)
    z2 = jnp.maximum(
        lax.dot_general(neigh, wn_ref[...], (((1,), (1,)), ((), ())),
                        preferred_element_type=jnp.float32), 0.0)
    o_ref[...] = jnp.maximum(z1 + z2, 0.0)


def kernel(h_neigh, h_self, edge_features, W_preagg, W_self, W_neigh,
           W_edge, b_edge, edge_index):
    src = edge_index[0]
    dst = edge_index[1]
    npad = E_PAD - E
    pad_dst = N + (jnp.arange(npad, dtype=jnp.int32) % (N_PAD - N))
    pad_src = jnp.arange(npad, dtype=jnp.int32) % N
    src_pad = jnp.concatenate(
        [src, pad_src]).reshape(NW * CH_PER_TILE, 1, CHUNK)
    dst_pad = jnp.concatenate(
        [dst, pad_dst]).reshape(NW * CH_PER_TILE, 1, CHUNK)
    hn = pl.pallas_call(
        _preagg_body,
        grid=(5,),
        in_specs=[pl.BlockSpec((2000, DIN), lambda i: (i, 0)),
                  pl.BlockSpec((HNW, DIN), lambda i: (0, 0))],
        out_specs=pl.BlockSpec((2000, HNW), lambda i: (i, 0)),
        out_shape=jax.ShapeDtypeStruct((N, HNW), jnp.float32),
    )(h_neigh, W_preagg)

    EB4 = 512
    NB4 = (E_PAD // 4) // EB4
    LASTB = (E - 1) // EB4

    def _mk(k):
        return lambda i: (0, jnp.minimum(i + NB4 * k, LASTB))

    ew = pl.pallas_call(
        _edge_body,
        grid=(NB4,),
        in_specs=[pl.BlockSpec((DE, EB4), _mk(0)),
                  pl.BlockSpec((DE, EB4), _mk(1)),
                  pl.BlockSpec((DE, EB4), _mk(2)),
                  pl.BlockSpec((DE, EB4), _mk(3)),
                  pl.BlockSpec((DOUT * DOUT, DE), lambda i: (0, 0)),
                  pl.BlockSpec((1, DOUT * DOUT), lambda i: (0, 0))],
        out_specs=pl.BlockSpec((EB4, 128), lambda i: (i, 0)),
        out_shape=jax.ShapeDtypeStruct((E_PAD // 4, 128), jnp.float32),
    )(edge_features.T, edge_features.T, edge_features.T, edge_features.T,
      W_edge.astype(jnp.bfloat16), b_edge.reshape(1, DOUT * DOUT))

    acc = _sc_edge_scatter(hn, src_pad, dst_pad, ew)

    z = pl.pallas_call(
        _final_body,
        grid=(10,),
        in_specs=[pl.BlockSpec((1000, DIN), lambda i: (i, 0)),
                  pl.BlockSpec((NC, 1000, AW), lambda i: (0, i, 0)),
                  pl.BlockSpec((DOUT, DIN), lambda i: (0, 0)),
                  pl.BlockSpec((DOUT, DOUT), lambda i: (0, 0)),
                  pl.BlockSpec((DOUT, DOUT), lambda i: (0, 0))],
        out_specs=pl.BlockSpec((1000, DOUT), lambda i: (i, 0)),
        out_shape=jax.ShapeDtypeStruct((N, DOUT), jnp.float32),
    )(h_self, acc, W_preagg, W_self, W_neigh)
    return z


# speedup vs baseline: 1.4731x; 1.0332x over previous
import functools

import jax
import jax.numpy as jnp
from jax import lax
from jax.experimental import pallas as pl
from jax.experimental.pallas import tpu as pltpu
from jax.experimental.pallas import tpu_sc as plsc

N = 10000
E = 50000
DIN = 256
DOUT = 32
DE = 16
HNW = DOUT

NC = 2
NS = 16
NW = NC * NS
CHUNK = 128
QPC = CHUNK // 4
CH_PER_TILE = 13
E_TILE = CHUNK * CH_PER_TILE
E_PAD = NW * E_TILE
N_PAD = 10240
ROWS_TILE = N_PAD // NS
AW = 48

_SC_MESH = plsc.VectorSubcoreMesh(
    core_axis_name="c", subcore_axis_name="s", num_cores=NC, num_subcores=NS)


@functools.partial(
    pl.kernel,
    out_type=jax.ShapeDtypeStruct((NC, N_PAD, AW), jnp.float32),
    mesh=_SC_MESH,
    compiler_params=pltpu.CompilerParams(use_tc_tiling_on_sc=False),
    scratch_types=[
        pltpu.VMEM((CH_PER_TILE, 1, CHUNK), jnp.int32),
        pltpu.VMEM((CH_PER_TILE, 1, CHUNK), jnp.int32),
        pltpu.VMEM((E_TILE, DOUT), jnp.float32),
        pltpu.VMEM((CHUNK, HNW), jnp.float32),
        pltpu.VMEM((CHUNK, HNW), jnp.float32),
        pltpu.VMEM((CHUNK, AW), jnp.float32),
        pltpu.VMEM((CHUNK, AW), jnp.float32),
        pltpu.VMEM_SHARED((N_PAD, AW), jnp.float32),
        pltpu.SemaphoreType.DMA,
        pltpu.SemaphoreType.DMA,
        pltpu.SemaphoreType.DMA,
        pltpu.SemaphoreType.DMA,
        pltpu.SemaphoreType.DMA,
    ],
)
def _sc_edge_scatter(hn_hbm, src_hbm, dst_hbm, ew_hbm, out_hbm,
                     idx_src, idx_dst, ew_all, rows0, rows1, msg0, msg1,
                     acc_sp, esem, gsem0, gsem1, ssem0, ssem1):
    c = lax.axis_index("c")
    s = lax.axis_index("s")
    wid = c * NS + s

    rows = (rows0, rows1)
    msg = (msg0, msg1)
    gsem = (gsem0, gsem1)
    ssem = (ssem0, ssem1)

    g = wid // 8
    ew_cp = pltpu.async_copy(
        ew_hbm.at[pl.ds((wid % 8) * E_TILE, E_TILE), pl.ds(g * DOUT, DOUT)],
        ew_all, esem)
    pltpu.sync_copy(src_hbm.at[pl.ds(wid * CH_PER_TILE, CH_PER_TILE)], idx_src)
    pltpu.sync_copy(dst_hbm.at[pl.ds(wid * CH_PER_TILE, CH_PER_TILE)], idx_dst)

    zeros16 = jnp.zeros((16,), jnp.float32)

    def _zero_row(i, _):
        msg0[i, pl.ds(0, 16)] = zeros16
        msg0[i, pl.ds(16, 16)] = zeros16
        msg0[i, pl.ds(32, 16)] = zeros16
        return 0
    lax.fori_loop(0, CHUNK, _zero_row, 0)

    zcps = [
        pltpu.async_copy(
            msg0, acc_sp.at[pl.ds(s * ROWS_TILE + j * CHUNK, CHUNK)],
            (gsem0, gsem1)[j & 1])
        for j in range(ROWS_TILE // CHUNK)
    ]
    for zc in zcps:
        zc.wait()

    deg_lane = jnp.where(lax.iota(jnp.int32, 16) == 0,
                         jnp.float32(1.0), jnp.float32(0.0))

    def _set_deg(i, _):
        msg0[i, pl.ds(DOUT, 16)] = deg_lane
        msg1[i, pl.ds(DOUT, 16)] = deg_lane
        return 0
    lax.fori_loop(0, CHUNK, _set_deg, 0)

    ew_cp.wait()
    plsc.subcore_barrier()

    gcp = [None, None]
    scp = [None, None]
    gcp[0] = pltpu.async_copy(hn_hbm.at[idx_src.at[0, 0]], rows0, gsem0)
    for j in range(CH_PER_TILE):
        b = j & 1
        nb = b ^ 1
        if j + 1 < CH_PER_TILE:
            gcp[nb] = pltpu.async_copy(
                hn_hbm.at[idx_src.at[j + 1, 0]], rows[nb], gsem[nb])
        gcp[b].wait()
        if scp[b] is not None:
            scp[b].wait()
        rb = rows[b]
        mb = msg[b]

        e0 = j * CHUNK

        @plsc.parallel_loop(0, QPC, unroll=2)
        def _mul(q):
            r = q * 4
            for k in range(4):
                for h in range(2):
                    mb[r + k, pl.ds(h * 16, 16)] = (
                        rb[r + k, pl.ds(h * 16, 16)]
                        * ew_all[e0 + r + k, pl.ds(h * 16, 16)])
        scp[b] = pltpu.async_copy(mb, acc_sp.at[idx_dst.at[j, 0]], ssem[b], add=True)
    scp[0].wait()
    scp[1].wait()
    plsc.subcore_barrier()

    wcp = [None, None]
    for j in range(ROWS_TILE // CHUNK):
        b = j & 1
        r0 = s * ROWS_TILE + j * CHUNK
        if wcp[b] is not None:
            wcp[b].wait()
        pltpu.async_copy(acc_sp.at[pl.ds(r0, CHUNK)], msg[b], gsem[b]).wait()
        wcp[b] = pltpu.async_copy(msg[b], out_hbm.at[c, pl.ds(r0, CHUNK)], ssem[b])
    for w in wcp:
        if w is not None:
            w.wait()


def _preagg_body(x0, x1, x2, x3, w_ref, o_ref):
    ys = []
    for x_ref in (x0, x1, x2, x3):
        y = lax.dot_general(x_ref[...], w_ref[...], (((1,), (1,)), ((), ())),
                            preferred_element_type=jnp.float32)
        ys.append(jnp.maximum(y, 0.0))
    o_ref[...] = jnp.concatenate(ys, axis=1)


def _edge_body(ef0, ef1, ef2, ef3, w_ref, b_ref, o_ref):
    ys = []
    for ef_ref in (ef0, ef1, ef2, ef3):
        y = lax.dot_general(ef_ref[...].astype(jnp.bfloat16), w_ref[...],
                            (((0,), (1,)), ((), ())),
                            preferred_element_type=jnp.float32)
        y = jnp.maximum(y + b_ref[...], 0.0)
        for half in (512, 256, 128, 64, 32):
            y = y[:, :half] + y[:, half:2 * half]
        ys.append(y)
    o_ref[...] = jnp.concatenate(ys, axis=1)


def _final_body(hs_ref, acc_ref, wp_ref, ws_ref, wn_ref, o_ref):
    hs = jnp.maximum(
        lax.dot_general(hs_ref[...], wp_ref[...], (((1,), (1,)), ((), ())),
                        preferred_element_type=jnp.float32), 0.0)
    a = acc_ref[0] + acc_ref[1]
    neigh = a[:, :DOUT] / jnp.maximum(a[:, DOUT:DOUT + 1], 1.0)
    z1 = jnp.maximum(
        lax.dot_general(hs, ws_ref[...], (((1,), (1,)), ((), ())),
                        preferred_element_type=jnp.float32), 0.0)
    z2 = jnp.maximum(
        lax.dot_general(neigh, wn_ref[...], (((1,), (1,)), ((), ())),
                        preferred_element_type=jnp.float32), 0.0)
    o_ref[...] = jnp.maximum(z1 + z2, 0.0)


def kernel(h_neigh, h_self, edge_features, W_preagg, W_self, W_neigh,
           W_edge, b_edge, edge_index):
    src = edge_index[0]
    dst = edge_index[1]
    npad = E_PAD - E
    pad_dst = N + (jnp.arange(npad, dtype=jnp.int32) % (N_PAD - N))
    pad_src = jnp.arange(npad, dtype=jnp.int32) % N
    src_all = jnp.concatenate([src, pad_src])
    src_pad = (4 * (src_all % (N_PAD // 4)) + src_all // (N_PAD // 4)).reshape(
        NW * CH_PER_TILE, 1, CHUNK)
    dst_pad = jnp.concatenate(
        [dst, pad_dst]).reshape(NW * CH_PER_TILE, 1, CHUNK)
    NQB = 5
    HQ = N_PAD // 4

    def _mkh(k):
        return lambda i: (i + NQB * k, 0)

    hn = pl.pallas_call(
        _preagg_body,
        grid=(NQB,),
        in_specs=[pl.BlockSpec((HQ // NQB, DIN), _mkh(0)),
                  pl.BlockSpec((HQ // NQB, DIN), _mkh(1)),
                  pl.BlockSpec((HQ // NQB, DIN), _mkh(2)),
                  pl.BlockSpec((HQ // NQB, DIN), _mkh(3)),
                  pl.BlockSpec((HNW, DIN), lambda i: (0, 0))],
        out_specs=pl.BlockSpec((HQ // NQB, 128), lambda i: (i, 0)),
        out_shape=jax.ShapeDtypeStruct((HQ, 128), jnp.float32),
    )(h_neigh, h_neigh, h_neigh, h_neigh, W_preagg).reshape(N_PAD, HNW)

    EB4 = 512
    NB4 = (E_PAD // 4) // EB4
    LASTB = (E - 1) // EB4

    def _mk(k):
        return lambda i: (0, jnp.minimum(i + NB4 * k, LASTB))

    ew = pl.pallas_call(
        _edge_body,
        grid=(NB4,),
        in_specs=[pl.BlockSpec((DE, EB4), _mk(0)),
                  pl.BlockSpec((DE, EB4), _mk(1)),
                  pl.BlockSpec((DE, EB4), _mk(2)),
                  pl.BlockSpec((DE, EB4), _mk(3)),
                  pl.BlockSpec((DOUT * DOUT, DE), lambda i: (0, 0)),
                  pl.BlockSpec((1, DOUT * DOUT), lambda i: (0, 0))],
        out_specs=pl.BlockSpec((EB4, 128), lambda i: (i, 0)),
        out_shape=jax.ShapeDtypeStruct((E_PAD // 4, 128), jnp.float32),
    )(edge_features.T, edge_features.T, edge_features.T, edge_features.T,
      W_edge.astype(jnp.bfloat16), b_edge.reshape(1, DOUT * DOUT))

    acc = _sc_edge_scatter(hn, src_pad, dst_pad, ew)

    z = pl.pallas_call(
        _final_body,
        grid=(10,),
        in_specs=[pl.BlockSpec((1000, DIN), lambda i: (i, 0)),
                  pl.BlockSpec((NC, 1000, AW), lambda i: (0, i, 0)),
                  pl.BlockSpec((DOUT, DIN), lambda i: (0, 0)),
                  pl.BlockSpec((DOUT, DOUT), lambda i: (0, 0)),
                  pl.BlockSpec((DOUT, DOUT), lambda i: (0, 0))],
        out_specs=pl.BlockSpec((1000, DOUT), lambda i: (i, 0)),
        out_shape=jax.ShapeDtypeStruct((N, DOUT), jnp.float32),
    )(h_self, acc, W_preagg, W_self, W_neigh)
    return z
